# baseline (speedup 1.0000x reference)
import functools

import jax
import jax.numpy as jnp
from jax import lax
from jax.experimental import pallas as pl
from jax.experimental.pallas import tpu as pltpu
from jax.experimental.pallas import tpu_sc as plsc

N_NODES = 10000
N_EDGES = 160000
IN_DIM = 256
HID = 512
LAT = 128
N_MAX = 128
NUM_GRAPHS = 64

NC = 2
NS = 16
NW = NC * NS
EPT = N_EDGES // NW
K = 125
STEPS = EPT // K
CHUNK = 128
RPT = 624
RTAIL = N_NODES - NS * RPT
RTOFF = NS * RPT

ROWBLK = 2000
NRB = N_NODES // ROWBLK


DEGW = 128


def _deg_sc_body(dstw, ones_h, zeros_h, out_h, idxd_v, ones_v, acc_sh):
    c = lax.axis_index("c")
    s = lax.axis_index("s")
    w = c * NS + s
    pltpu.sync_copy(dstw.at[w], idxd_v)
    pltpu.sync_copy(ones_h, ones_v)
    r0 = s * RPT
    pltpu.sync_copy(zeros_h, acc_sh.at[pl.ds(r0, RPT)])

    @pl.when(s == 0)
    def _():
        pltpu.sync_copy(zeros_h.at[pl.ds(0, RTAIL)],
                        acc_sh.at[pl.ds(RTOFF, RTAIL)])

    plsc.subcore_barrier()

    def step(j, carry):
        pltpu.sync_copy(ones_v, acc_sh.at[idxd_v.at[j]], add=True)
        return carry

    lax.fori_loop(0, STEPS, step, 0)
    plsc.subcore_barrier()
    pltpu.sync_copy(acc_sh.at[pl.ds(r0, RPT)], out_h.at[c].at[pl.ds(r0, RPT)])

    @pl.when(s == 0)
    def _():
        pltpu.sync_copy(acc_sh.at[pl.ds(RTOFF, RTAIL)],
                        out_h.at[c].at[pl.ds(RTOFF, RTAIL)])


_deg_call = pl.kernel(
    _deg_sc_body,
    out_type=jax.ShapeDtypeStruct((NC, N_NODES, DEGW), jnp.float32),
    mesh=plsc.VectorSubcoreMesh(core_axis_name="c", subcore_axis_name="s"),
    scratch_types=[
        pltpu.VMEM((STEPS, K), jnp.int32),
        pltpu.VMEM((K, DEGW), jnp.float32),
        pltpu.VMEM_SHARED((N_NODES, DEGW), jnp.float32),
    ],
)


NPAIR = STEPS // 2


def _agg_sc_body(xk_h, srcw, dstw, zeros_h, out_h, idxs_v, idxd_v,
                 buf0, buf1, sem0, sem1, acc_sh):
    c = lax.axis_index("c")
    s = lax.axis_index("s")
    w = c * NS + s
    pltpu.sync_copy(srcw.at[w], idxs_v)
    pltpu.sync_copy(dstw.at[w], idxd_v)
    r0 = s * RPT
    pltpu.sync_copy(zeros_h, acc_sh.at[pl.ds(r0, RPT)])

    @pl.when(s == 0)
    def _():
        pltpu.sync_copy(zeros_h.at[pl.ds(0, RTAIL)],
                        acc_sh.at[pl.ds(RTOFF, RTAIL)])

    plsc.subcore_barrier()
    pltpu.async_copy(xk_h.at[idxs_v.at[0]], buf0, sem0)

    def pair(j2, carry):
        j = j2 * 2
        pltpu.make_async_copy(xk_h.at[idxs_v.at[j]], buf0, sem0).wait()
        pltpu.async_copy(xk_h.at[idxs_v.at[j + 1]], buf1, sem1)
        pltpu.sync_copy(buf0, acc_sh.at[idxd_v.at[j]], add=True)
        pltpu.make_async_copy(xk_h.at[idxs_v.at[j + 1]], buf1, sem1).wait()

        @pl.when(j2 + 1 < NPAIR)
        def _():
            pltpu.async_copy(xk_h.at[idxs_v.at[j + 2]], buf0, sem0)

        pltpu.sync_copy(buf1, acc_sh.at[idxd_v.at[j + 1]], add=True)
        return carry

    lax.fori_loop(0, NPAIR, pair, 0)
    plsc.subcore_barrier()
    pltpu.sync_copy(acc_sh.at[pl.ds(r0, RPT)], out_h.at[c].at[pl.ds(r0, RPT)])

    @pl.when(s == 0)
    def _():
        pltpu.sync_copy(acc_sh.at[pl.ds(RTOFF, RTAIL)],
                        out_h.at[c].at[pl.ds(RTOFF, RTAIL)])


_agg_call = pl.kernel(
    _agg_sc_body,
    out_type=jax.ShapeDtypeStruct((NC, N_NODES, CHUNK), jnp.float32),
    mesh=plsc.VectorSubcoreMesh(core_axis_name="c", subcore_axis_name="s"),
    scratch_types=[
        pltpu.VMEM((STEPS, K), jnp.int32),
        pltpu.VMEM((STEPS, K), jnp.int32),
        pltpu.VMEM((K, CHUNK), jnp.float32),
        pltpu.VMEM((K, CHUNK), jnp.float32),
        pltpu.SemaphoreType.DMA,
        pltpu.SemaphoreType.DMA,
        pltpu.VMEM_SHARED((N_NODES, CHUNK), jnp.float32),
    ],
)


def _scale_body(x_ref, dinv_ref, xs_ref):
    dv = dinv_ref[...]
    xv = x_ref[...]
    for k in range(IN_DIM // CHUNK):
        xs_ref[k] = xv[:, k * CHUNK:(k + 1) * CHUNK] * dv


def _conv1_body(a0_ref, a1_ref, x_ref, dinv_ref, w1_ref, b1_ref,
                h1_ref, hs_ref):
    a = jnp.concatenate([a0_ref[0] + a0_ref[1], a1_ref[0] + a1_ref[1]], axis=1)
    dv = dinv_ref[...]
    t = a * dv + x_ref[...] * (dv * dv)
    h1 = jnp.maximum(
        jnp.dot(t, w1_ref[...], preferred_element_type=jnp.float32)
        + b1_ref[...], 0.0)
    h1_ref[...] = h1
    hs = h1 * dv
    for k in range(HID // CHUNK):
        hs_ref[k] = hs[:, k * CHUNK:(k + 1) * CHUNK]


def _conv2_pool_body(a0_ref, a1_ref, a2_ref, a3_ref, h1_ref, dinv_ref,
                     batch_ref, w2_ref, b2_ref, gsum_ref, cnt_ref):
    i = pl.program_id(0)
    a = jnp.concatenate(
        [a0_ref[0] + a0_ref[1], a1_ref[0] + a1_ref[1],
         a2_ref[0] + a2_ref[1], a3_ref[0] + a3_ref[1]], axis=1)
    dv = dinv_ref[...]
    t = a * dv + h1_ref[...] * (dv * dv)
    h2 = jnp.maximum(
        jnp.dot(t, w2_ref[...], preferred_element_type=jnp.float32)
        + b2_ref[...], 0.0)
    gids = lax.broadcasted_iota(jnp.int32, (ROWBLK, NUM_GRAPHS), 1)
    oh = (batch_ref[...] == gids).astype(jnp.float32)
    gp = lax.dot_general(oh, h2, (((0,), (0,)), ((), ())),
                         preferred_element_type=jnp.float32)
    cp = jnp.sum(oh, axis=0, keepdims=True)

    @pl.when(i == 0)
    def _():
        gsum_ref[...] = jnp.zeros_like(gsum_ref)
        cnt_ref[...] = jnp.zeros_like(cnt_ref)

    gsum_ref[...] += gp
    cnt_ref[...] += cp


def _head_body(gsum_ref, cnt_ref, wmu_ref, bmu_ref, wlv_ref, blv_ref, eps_ref,
               wd1_ref, bd1_ref, wd2_ref, bd2_ref,
               mu_ref, lv_ref, h2_ref):
    cnt = jnp.maximum(cnt_ref[...], 1.0)
    g = gsum_ref[...] / jnp.transpose(cnt)
    mu = jnp.dot(g, wmu_ref[...], preferred_element_type=jnp.float32) + bmu_ref[...]
    lv = jnp.dot(g, wlv_ref[...], preferred_element_type=jnp.float32) + blv_ref[...]
    mu_ref[...] = mu
    lv_ref[...] = lv
    z = mu + jnp.exp(0.5 * lv) * eps_ref[...]
    h = jnp.maximum(jnp.dot(z, wd1_ref[...], preferred_element_type=jnp.float32)
                    + bd1_ref[...], 0.0)
    h = jnp.maximum(jnp.dot(h, wd2_ref[...], preferred_element_type=jnp.float32)
                    + bd2_ref[...], 0.0)
    h2_ref[...] = h


_LCOLS = 4096


def _logits_body(h2_ref, wd3_ref, bd3_ref, out_ref):
    b = pl.program_id(0)
    val = (jnp.dot(h2_ref[...], wd3_ref[...],
                   preferred_element_type=jnp.float32) + bd3_ref[...])
    flat = b * _LCOLS + lax.broadcasted_iota(jnp.int32, (NUM_GRAPHS, _LCOLS), 1)
    diag = (flat // N_MAX) == (flat % N_MAX)
    out_ref[...] = jnp.where(diag, 0.0, 1.0 / (1.0 + jnp.exp(-val)))


def kernel(x, edge_index, batch, W1, b1, W2, b2, Wmu, bmu, Wlv, blv,
           Wd1, bd1, Wd2, bd2, Wd3, bd3):
    f32 = jnp.float32
    src2 = edge_index[0].reshape(NW, STEPS, K)
    dst2 = edge_index[1].reshape(NW, STEPS, K)
    ones_k = jnp.ones((K, DEGW), f32)
    zeros_small = jnp.zeros((RPT, CHUNK), f32)

    degp = _deg_call(dst2, ones_k, zeros_small)
    dinv = lax.rsqrt(1.0 + degp[0, :, :1] + degp[1, :, :1])

    xs_chunks = pl.pallas_call(
        _scale_body,
        grid=(NRB,),
        in_specs=[
            pl.BlockSpec((ROWBLK, IN_DIM), lambda i: (i, 0)),
            pl.BlockSpec((ROWBLK, 1), lambda i: (i, 0)),
        ],
        out_specs=pl.BlockSpec((IN_DIM // CHUNK, ROWBLK, CHUNK), lambda i: (0, i, 0)),
        out_shape=jax.ShapeDtypeStruct((IN_DIM // CHUNK, N_NODES, CHUNK), f32),
    )(x, dinv)

    srcs = [src2 + jnp.int32(k * N_NODES) for k in range(HID // CHUNK)]
    xs_flat = xs_chunks.reshape((IN_DIM // CHUNK) * N_NODES, CHUNK)

    a1 = [_agg_call(xs_flat, srcs[k], dst2, zeros_small)
          for k in range(IN_DIM // CHUNK)]

    h1, hs_chunks = pl.pallas_call(
        _conv1_body,
        grid=(NRB,),
        in_specs=[
            pl.BlockSpec((NC, ROWBLK, CHUNK), lambda i: (0, i, 0)),
            pl.BlockSpec((NC, ROWBLK, CHUNK), lambda i: (0, i, 0)),
            pl.BlockSpec((ROWBLK, IN_DIM), lambda i: (i, 0)),
            pl.BlockSpec((ROWBLK, 1), lambda i: (i, 0)),
            pl.BlockSpec((IN_DIM, HID), lambda i: (0, 0)),
            pl.BlockSpec((HID,), lambda i: (0,)),
        ],
        out_specs=(
            pl.BlockSpec((ROWBLK, HID), lambda i: (i, 0)),
            pl.BlockSpec((HID // CHUNK, ROWBLK, CHUNK), lambda i: (0, i, 0)),
        ),
        out_shape=(
            jax.ShapeDtypeStruct((N_NODES, HID), f32),
            jax.ShapeDtypeStruct((HID // CHUNK, N_NODES, CHUNK), f32),
        ),
    )(a1[0], a1[1], x, dinv, W1, b1)

    hs_flat = hs_chunks.reshape((HID // CHUNK) * N_NODES, CHUNK)
    a2 = [_agg_call(hs_flat, srcs[k], dst2, zeros_small)
          for k in range(HID // CHUNK)]

    gsum, cnt = pl.pallas_call(
        _conv2_pool_body,
        grid=(NRB,),
        in_specs=[
            pl.BlockSpec((NC, ROWBLK, CHUNK), lambda i: (0, i, 0)),
            pl.BlockSpec((NC, ROWBLK, CHUNK), lambda i: (0, i, 0)),
            pl.BlockSpec((NC, ROWBLK, CHUNK), lambda i: (0, i, 0)),
            pl.BlockSpec((NC, ROWBLK, CHUNK), lambda i: (0, i, 0)),
            pl.BlockSpec((ROWBLK, HID), lambda i: (i, 0)),
            pl.BlockSpec((ROWBLK, 1), lambda i: (i, 0)),
            pl.BlockSpec((ROWBLK, 1), lambda i: (i, 0)),
            pl.BlockSpec((HID, HID), lambda i: (0, 0)),
            pl.BlockSpec((HID,), lambda i: (0,)),
        ],
        out_specs=(
            pl.BlockSpec((NUM_GRAPHS, HID), lambda i: (0, 0)),
            pl.BlockSpec((1, NUM_GRAPHS), lambda i: (0, 0)),
        ),
        out_shape=(
            jax.ShapeDtypeStruct((NUM_GRAPHS, HID), f32),
            jax.ShapeDtypeStruct((1, NUM_GRAPHS), f32),
        ),
    )(a2[0], a2[1], a2[2], a2[3], h1, dinv, batch[:, None], W2, b2)

    eps = jax.random.normal(jax.random.key(42), (NUM_GRAPHS, LAT), dtype=f32)

    mu, logv, hdec = pl.pallas_call(
        _head_body,
        out_shape=(
            jax.ShapeDtypeStruct((NUM_GRAPHS, LAT), f32),
            jax.ShapeDtypeStruct((NUM_GRAPHS, LAT), f32),
            jax.ShapeDtypeStruct((NUM_GRAPHS, HID), f32),
        ),
    )(gsum, cnt, Wmu, bmu, Wlv, blv, eps, Wd1, bd1, Wd2, bd2)

    w3 = Wd3.reshape(HID, N_MAX, N_MAX)
    Wsym = ((w3 + jnp.transpose(w3, (0, 2, 1))) * 0.5).reshape(HID, N_MAX * N_MAX)
    b3 = bd3.reshape(N_MAX, N_MAX)
    bsym = ((b3 + b3.T) * 0.5).reshape(N_MAX * N_MAX)

    probs = pl.pallas_call(
        _logits_body,
        grid=((N_MAX * N_MAX) // _LCOLS,),
        in_specs=[
            pl.BlockSpec((NUM_GRAPHS, HID), lambda i: (0, 0)),
            pl.BlockSpec((HID, _LCOLS), lambda i: (0, i)),
            pl.BlockSpec((_LCOLS,), lambda i: (i,)),
        ],
        out_specs=pl.BlockSpec((NUM_GRAPHS, _LCOLS), lambda i: (0, i)),
        out_shape=jax.ShapeDtypeStruct((NUM_GRAPHS, N_MAX * N_MAX), f32),
    )(hdec, Wsym, bsym)

    return (probs.reshape(NUM_GRAPHS, N_MAX, N_MAX), mu, logv)


# speedup vs baseline: 1.1365x; 1.1365x over previous
import functools

import jax
import jax.numpy as jnp
from jax import lax
from jax.experimental import pallas as pl
from jax.experimental.pallas import tpu as pltpu
from jax.experimental.pallas import tpu_sc as plsc

N_NODES = 10000
N_EDGES = 160000
IN_DIM = 256
HID = 512
LAT = 128
N_MAX = 128
NUM_GRAPHS = 64

NC = 2
NS = 16
NW = NC * NS
EPT = N_EDGES // NW
K = 125
STEPS = EPT // K
CHUNK = 128
RPT = 624
RTAIL = N_NODES - NS * RPT
RTOFF = NS * RPT

ROWBLK = 2000
NRB = N_NODES // ROWBLK


DEGW = 128


def _deg_sc_body(dstw, ones_h, zeros_h, out_h, idxd_v, ones_v, acc_sh):
    c = lax.axis_index("c")
    s = lax.axis_index("s")
    w = c * NS + s
    pltpu.sync_copy(dstw.at[w], idxd_v)
    pltpu.sync_copy(ones_h, ones_v)
    r0 = s * RPT
    pltpu.sync_copy(zeros_h, acc_sh.at[pl.ds(r0, RPT)])

    @pl.when(s == 0)
    def _():
        pltpu.sync_copy(zeros_h.at[pl.ds(0, RTAIL)],
                        acc_sh.at[pl.ds(RTOFF, RTAIL)])

    plsc.subcore_barrier()

    def step(j, carry):
        pltpu.sync_copy(ones_v, acc_sh.at[idxd_v.at[j]], add=True)
        return carry

    lax.fori_loop(0, STEPS, step, 0)
    plsc.subcore_barrier()
    pltpu.sync_copy(acc_sh.at[pl.ds(r0, RPT)], out_h.at[c].at[pl.ds(r0, RPT)])

    @pl.when(s == 0)
    def _():
        pltpu.sync_copy(acc_sh.at[pl.ds(RTOFF, RTAIL)],
                        out_h.at[c].at[pl.ds(RTOFF, RTAIL)])


_deg_call = pl.kernel(
    _deg_sc_body,
    out_type=jax.ShapeDtypeStruct((NC, N_NODES, DEGW), jnp.float32),
    mesh=plsc.VectorSubcoreMesh(core_axis_name="c", subcore_axis_name="s"),
    scratch_types=[
        pltpu.VMEM((STEPS, K), jnp.int32),
        pltpu.VMEM((K, DEGW), jnp.float32),
        pltpu.VMEM_SHARED((N_NODES, DEGW), jnp.float32),
    ],
)


NPAIR = STEPS // 2


def _agg_sc_body(xk_h, srcw, dstw, zeros_h, out_h, idxs_v, idxd_v,
                 buf0, buf1, sem0, sem1, acc_sh):
    c = lax.axis_index("c")
    s = lax.axis_index("s")
    w = c * NS + s
    pltpu.sync_copy(srcw.at[w], idxs_v)
    pltpu.sync_copy(dstw.at[w], idxd_v)
    r0 = s * RPT
    pltpu.sync_copy(zeros_h, acc_sh.at[pl.ds(r0, RPT)])

    @pl.when(s == 0)
    def _():
        pltpu.sync_copy(zeros_h.at[pl.ds(0, RTAIL)],
                        acc_sh.at[pl.ds(RTOFF, RTAIL)])

    plsc.subcore_barrier()
    pltpu.async_copy(xk_h.at[idxs_v.at[0]], buf0, sem0)

    def pair(j2, carry):
        j = j2 * 2
        pltpu.make_async_copy(xk_h.at[idxs_v.at[j]], buf0, sem0).wait()
        pltpu.async_copy(xk_h.at[idxs_v.at[j + 1]], buf1, sem1)
        pltpu.sync_copy(buf0, acc_sh.at[idxd_v.at[j]], add=True)
        pltpu.make_async_copy(xk_h.at[idxs_v.at[j + 1]], buf1, sem1).wait()

        @pl.when(j2 + 1 < NPAIR)
        def _():
            pltpu.async_copy(xk_h.at[idxs_v.at[j + 2]], buf0, sem0)

        pltpu.sync_copy(buf1, acc_sh.at[idxd_v.at[j + 1]], add=True)
        return carry

    lax.fori_loop(0, NPAIR, pair, 0)
    plsc.subcore_barrier()
    pltpu.sync_copy(acc_sh.at[pl.ds(r0, RPT)], out_h.at[c].at[pl.ds(r0, RPT)])

    @pl.when(s == 0)
    def _():
        pltpu.sync_copy(acc_sh.at[pl.ds(RTOFF, RTAIL)],
                        out_h.at[c].at[pl.ds(RTOFF, RTAIL)])


_agg_call = pl.kernel(
    _agg_sc_body,
    out_type=jax.ShapeDtypeStruct((NC, N_NODES, CHUNK), jnp.float32),
    mesh=plsc.VectorSubcoreMesh(core_axis_name="c", subcore_axis_name="s"),
    scratch_types=[
        pltpu.VMEM((STEPS, K), jnp.int32),
        pltpu.VMEM((STEPS, K), jnp.int32),
        pltpu.VMEM((K, CHUNK), jnp.float32),
        pltpu.VMEM((K, CHUNK), jnp.float32),
        pltpu.SemaphoreType.DMA,
        pltpu.SemaphoreType.DMA,
        pltpu.VMEM_SHARED((N_NODES, CHUNK), jnp.float32),
    ],
)


def _scale_body(x_ref, dinv_ref, xs_ref):
    dv = dinv_ref[...]
    xv = x_ref[...]
    for k in range(IN_DIM // CHUNK):
        xs_ref[k] = xv[:, k * CHUNK:(k + 1) * CHUNK] * dv


def _conv1_body(a0_ref, a1_ref, x_ref, dinv_ref, w1_ref, b1_ref,
                h1_ref, hs_ref):
    a = jnp.concatenate([a0_ref[0] + a0_ref[1], a1_ref[0] + a1_ref[1]], axis=1)
    dv = dinv_ref[...]
    t = a * dv + x_ref[...] * (dv * dv)
    h1 = jnp.maximum(
        jnp.dot(t, w1_ref[...], preferred_element_type=jnp.float32)
        + b1_ref[...], 0.0)
    h1_ref[...] = h1
    hs = h1 * dv
    for k in range(HID // CHUNK):
        hs_ref[k] = hs[:, k * CHUNK:(k + 1) * CHUNK]


def _conv2_pool_body(a0_ref, a1_ref, a2_ref, a3_ref, h1_ref, dinv_ref,
                     batch_ref, w2_ref, b2_ref, gsum_ref, cnt_ref):
    i = pl.program_id(0)
    a = jnp.concatenate(
        [a0_ref[0] + a0_ref[1], a1_ref[0] + a1_ref[1],
         a2_ref[0] + a2_ref[1], a3_ref[0] + a3_ref[1]], axis=1)
    dv = dinv_ref[...]
    t = a * dv + h1_ref[...] * (dv * dv)
    h2 = jnp.maximum(
        jnp.dot(t, w2_ref[...], preferred_element_type=jnp.float32)
        + b2_ref[...], 0.0)
    gids = lax.broadcasted_iota(jnp.int32, (ROWBLK, NUM_GRAPHS), 1)
    oh = (batch_ref[...] == gids).astype(jnp.float32)
    gp = lax.dot_general(oh, h2, (((0,), (0,)), ((), ())),
                         preferred_element_type=jnp.float32)
    cp = jnp.sum(oh, axis=0, keepdims=True)

    @pl.when(i == 0)
    def _():
        gsum_ref[...] = jnp.zeros_like(gsum_ref)
        cnt_ref[...] = jnp.zeros_like(cnt_ref)

    gsum_ref[...] += gp
    cnt_ref[...] += cp


def _head_body(gsum_ref, cnt_ref, wmu_ref, bmu_ref, wlv_ref, blv_ref, eps_ref,
               wd1_ref, bd1_ref, wd2_ref, bd2_ref,
               mu_ref, lv_ref, h2_ref):
    cnt = jnp.maximum(cnt_ref[...], 1.0)
    g = gsum_ref[...] / jnp.transpose(cnt)
    mu = jnp.dot(g, wmu_ref[...], preferred_element_type=jnp.float32) + bmu_ref[...]
    lv = jnp.dot(g, wlv_ref[...], preferred_element_type=jnp.float32) + blv_ref[...]
    mu_ref[...] = mu
    lv_ref[...] = lv
    z = mu + jnp.exp(0.5 * lv) * eps_ref[...]
    h = jnp.maximum(jnp.dot(z, wd1_ref[...], preferred_element_type=jnp.float32)
                    + bd1_ref[...], 0.0)
    h = jnp.maximum(jnp.dot(h, wd2_ref[...], preferred_element_type=jnp.float32)
                    + bd2_ref[...], 0.0)
    h2_ref[...] = h


_LCOLS = 4096
_LROWS = _LCOLS // N_MAX


def _logits_body(h2_ref, wd3_ref, bd3_ref, out_ref):
    val = (jnp.dot(h2_ref[...], wd3_ref[...],
                   preferred_element_type=jnp.float32) + bd3_ref[...])
    out_ref[...] = val.reshape(NUM_GRAPHS, _LROWS, N_MAX)


def _sym_body(l_ref, eye_ref, out_ref):
    l = l_ref[...]
    lt = lax.dot_general(l, eye_ref[...], (((1,), (0,)), ((), ())),
                         preferred_element_type=jnp.float32)
    ls = (l + lt) * 0.5
    r = lax.broadcasted_iota(jnp.int32, (NUM_GRAPHS, N_MAX, N_MAX), 1)
    c = lax.broadcasted_iota(jnp.int32, (NUM_GRAPHS, N_MAX, N_MAX), 2)
    out_ref[...] = jnp.where(r == c, 0.0, 1.0 / (1.0 + jnp.exp(-ls)))


def kernel(x, edge_index, batch, W1, b1, W2, b2, Wmu, bmu, Wlv, blv,
           Wd1, bd1, Wd2, bd2, Wd3, bd3):
    f32 = jnp.float32
    src2 = edge_index[0].reshape(NW, STEPS, K)
    dst2 = edge_index[1].reshape(NW, STEPS, K)
    ones_k = jnp.ones((K, DEGW), f32)
    zeros_small = jnp.zeros((RPT, CHUNK), f32)

    degp = _deg_call(dst2, ones_k, zeros_small)
    dinv = lax.rsqrt(1.0 + degp[0, :, :1] + degp[1, :, :1])

    xs_chunks = pl.pallas_call(
        _scale_body,
        grid=(NRB,),
        in_specs=[
            pl.BlockSpec((ROWBLK, IN_DIM), lambda i: (i, 0)),
            pl.BlockSpec((ROWBLK, 1), lambda i: (i, 0)),
        ],
        out_specs=pl.BlockSpec((IN_DIM // CHUNK, ROWBLK, CHUNK), lambda i: (0, i, 0)),
        out_shape=jax.ShapeDtypeStruct((IN_DIM // CHUNK, N_NODES, CHUNK), f32),
    )(x, dinv)

    srcs = [src2 + jnp.int32(k * N_NODES) for k in range(HID // CHUNK)]
    xs_flat = xs_chunks.reshape((IN_DIM // CHUNK) * N_NODES, CHUNK)

    a1 = [_agg_call(xs_flat, srcs[k], dst2, zeros_small)
          for k in range(IN_DIM // CHUNK)]

    h1, hs_chunks = pl.pallas_call(
        _conv1_body,
        grid=(NRB,),
        in_specs=[
            pl.BlockSpec((NC, ROWBLK, CHUNK), lambda i: (0, i, 0)),
            pl.BlockSpec((NC, ROWBLK, CHUNK), lambda i: (0, i, 0)),
            pl.BlockSpec((ROWBLK, IN_DIM), lambda i: (i, 0)),
            pl.BlockSpec((ROWBLK, 1), lambda i: (i, 0)),
            pl.BlockSpec((IN_DIM, HID), lambda i: (0, 0)),
            pl.BlockSpec((HID,), lambda i: (0,)),
        ],
        out_specs=(
            pl.BlockSpec((ROWBLK, HID), lambda i: (i, 0)),
            pl.BlockSpec((HID // CHUNK, ROWBLK, CHUNK), lambda i: (0, i, 0)),
        ),
        out_shape=(
            jax.ShapeDtypeStruct((N_NODES, HID), f32),
            jax.ShapeDtypeStruct((HID // CHUNK, N_NODES, CHUNK), f32),
        ),
    )(a1[0], a1[1], x, dinv, W1, b1)

    hs_flat = hs_chunks.reshape((HID // CHUNK) * N_NODES, CHUNK)
    a2 = [_agg_call(hs_flat, srcs[k], dst2, zeros_small)
          for k in range(HID // CHUNK)]

    gsum, cnt = pl.pallas_call(
        _conv2_pool_body,
        grid=(NRB,),
        in_specs=[
            pl.BlockSpec((NC, ROWBLK, CHUNK), lambda i: (0, i, 0)),
            pl.BlockSpec((NC, ROWBLK, CHUNK), lambda i: (0, i, 0)),
            pl.BlockSpec((NC, ROWBLK, CHUNK), lambda i: (0, i, 0)),
            pl.BlockSpec((NC, ROWBLK, CHUNK), lambda i: (0, i, 0)),
            pl.BlockSpec((ROWBLK, HID), lambda i: (i, 0)),
            pl.BlockSpec((ROWBLK, 1), lambda i: (i, 0)),
            pl.BlockSpec((ROWBLK, 1), lambda i: (i, 0)),
            pl.BlockSpec((HID, HID), lambda i: (0, 0)),
            pl.BlockSpec((HID,), lambda i: (0,)),
        ],
        out_specs=(
            pl.BlockSpec((NUM_GRAPHS, HID), lambda i: (0, 0)),
            pl.BlockSpec((1, NUM_GRAPHS), lambda i: (0, 0)),
        ),
        out_shape=(
            jax.ShapeDtypeStruct((NUM_GRAPHS, HID), f32),
            jax.ShapeDtypeStruct((1, NUM_GRAPHS), f32),
        ),
    )(a2[0], a2[1], a2[2], a2[3], h1, dinv, batch[:, None], W2, b2)

    eps = jax.random.normal(jax.random.key(42), (NUM_GRAPHS, LAT), dtype=f32)

    mu, logv, hdec = pl.pallas_call(
        _head_body,
        out_shape=(
            jax.ShapeDtypeStruct((NUM_GRAPHS, LAT), f32),
            jax.ShapeDtypeStruct((NUM_GRAPHS, LAT), f32),
            jax.ShapeDtypeStruct((NUM_GRAPHS, HID), f32),
        ),
    )(gsum, cnt, Wmu, bmu, Wlv, blv, eps, Wd1, bd1, Wd2, bd2)

    logits = pl.pallas_call(
        _logits_body,
        grid=((N_MAX * N_MAX) // _LCOLS,),
        in_specs=[
            pl.BlockSpec((NUM_GRAPHS, HID), lambda i: (0, 0)),
            pl.BlockSpec((HID, _LCOLS), lambda i: (0, i)),
            pl.BlockSpec((_LCOLS,), lambda i: (i,)),
        ],
        out_specs=pl.BlockSpec((NUM_GRAPHS, _LROWS, N_MAX), lambda i: (0, i, 0)),
        out_shape=jax.ShapeDtypeStruct((NUM_GRAPHS, N_MAX, N_MAX), f32),
    )(hdec, Wd3, bd3)

    probs = pl.pallas_call(
        _sym_body,
        out_shape=jax.ShapeDtypeStruct((NUM_GRAPHS, N_MAX, N_MAX), f32),
    )(logits, jnp.eye(N_MAX, dtype=f32))

    return (probs, mu, logv)


# speedup vs baseline: 1.1408x; 1.0038x over previous
import functools

import jax
import jax.numpy as jnp
from jax import lax
from jax.experimental import pallas as pl
from jax.experimental.pallas import tpu as pltpu
from jax.experimental.pallas import tpu_sc as plsc

N_NODES = 10000
N_EDGES = 160000
IN_DIM = 256
HID = 512
LAT = 128
N_MAX = 128
NUM_GRAPHS = 64

NC = 2
NS = 16
NW = NC * NS
EPT = N_EDGES // NW
K = 125
STEPS = EPT // K
CHUNK = 128
RPT = 624
RTAIL = N_NODES - NS * RPT
RTOFF = NS * RPT

ROWBLK = 2000
NRB = N_NODES // ROWBLK


DEGW = 128


def _deg_sc_body(dstw, ones_h, zeros_h, out_h, idxd_v, ones_v, acc_sh):
    c = lax.axis_index("c")
    s = lax.axis_index("s")
    w = c * NS + s
    pltpu.sync_copy(dstw.at[w], idxd_v)
    pltpu.sync_copy(ones_h, ones_v)
    r0 = s * RPT
    pltpu.sync_copy(zeros_h, acc_sh.at[pl.ds(r0, RPT)])

    @pl.when(s == 0)
    def _():
        pltpu.sync_copy(zeros_h.at[pl.ds(0, RTAIL)],
                        acc_sh.at[pl.ds(RTOFF, RTAIL)])

    plsc.subcore_barrier()

    def step(j, carry):
        pltpu.sync_copy(ones_v, acc_sh.at[idxd_v.at[j]], add=True)
        return carry

    lax.fori_loop(0, STEPS, step, 0)
    plsc.subcore_barrier()
    pltpu.sync_copy(acc_sh.at[pl.ds(r0, RPT)], out_h.at[c].at[pl.ds(r0, RPT)])

    @pl.when(s == 0)
    def _():
        pltpu.sync_copy(acc_sh.at[pl.ds(RTOFF, RTAIL)],
                        out_h.at[c].at[pl.ds(RTOFF, RTAIL)])


_deg_call = pl.kernel(
    _deg_sc_body,
    out_type=jax.ShapeDtypeStruct((NC, N_NODES, DEGW), jnp.float32),
    mesh=plsc.VectorSubcoreMesh(core_axis_name="c", subcore_axis_name="s"),
    scratch_types=[
        pltpu.VMEM((STEPS, K), jnp.int32),
        pltpu.VMEM((K, DEGW), jnp.float32),
        pltpu.VMEM_SHARED((N_NODES, DEGW), jnp.float32),
    ],
)


NPAIR = STEPS // 2


def _agg_sc_body(xk_h, srcw, dstw, zeros_h, out_h, idxs_v, idxd_v,
                 buf0, buf1, sem0, sem1, acc_sh):
    c = lax.axis_index("c")
    s = lax.axis_index("s")
    w = c * NS + s
    pltpu.sync_copy(srcw.at[w], idxs_v)
    pltpu.sync_copy(dstw.at[w], idxd_v)
    r0 = s * RPT
    pltpu.sync_copy(zeros_h, acc_sh.at[pl.ds(r0, RPT)])

    @pl.when(s == 0)
    def _():
        pltpu.sync_copy(zeros_h.at[pl.ds(0, RTAIL)],
                        acc_sh.at[pl.ds(RTOFF, RTAIL)])

    plsc.subcore_barrier()
    pltpu.async_copy(xk_h.at[idxs_v.at[0]], buf0, sem0)

    def pair(j2, carry):
        j = j2 * 2
        pltpu.make_async_copy(xk_h.at[idxs_v.at[j]], buf0, sem0).wait()
        pltpu.async_copy(xk_h.at[idxs_v.at[j + 1]], buf1, sem1)
        pltpu.sync_copy(buf0, acc_sh.at[idxd_v.at[j]], add=True)
        pltpu.make_async_copy(xk_h.at[idxs_v.at[j + 1]], buf1, sem1).wait()

        @pl.when(j2 + 1 < NPAIR)
        def _():
            pltpu.async_copy(xk_h.at[idxs_v.at[j + 2]], buf0, sem0)

        pltpu.sync_copy(buf1, acc_sh.at[idxd_v.at[j + 1]], add=True)
        return carry

    lax.fori_loop(0, NPAIR, pair, 0)
    plsc.subcore_barrier()
    pltpu.sync_copy(acc_sh.at[pl.ds(r0, RPT)], out_h.at[c].at[pl.ds(r0, RPT)])

    @pl.when(s == 0)
    def _():
        pltpu.sync_copy(acc_sh.at[pl.ds(RTOFF, RTAIL)],
                        out_h.at[c].at[pl.ds(RTOFF, RTAIL)])


_agg_call = pl.kernel(
    _agg_sc_body,
    out_type=jax.ShapeDtypeStruct((NC, N_NODES, CHUNK), jnp.float32),
    mesh=plsc.VectorSubcoreMesh(core_axis_name="c", subcore_axis_name="s"),
    scratch_types=[
        pltpu.VMEM((STEPS, K), jnp.int32),
        pltpu.VMEM((STEPS, K), jnp.int32),
        pltpu.VMEM((K, CHUNK), jnp.float32),
        pltpu.VMEM((K, CHUNK), jnp.float32),
        pltpu.SemaphoreType.DMA,
        pltpu.SemaphoreType.DMA,
        pltpu.VMEM_SHARED((N_NODES, CHUNK), jnp.float32),
    ],
)


def _scale_body(degp_ref, x_ref, xs_ref, dinv_ref):
    p = degp_ref[...]
    dv = lax.rsqrt(1.0 + p[0, :, 0:1] + p[1, :, 0:1])
    dinv_ref[...] = dv
    xv = x_ref[...]
    for k in range(IN_DIM // CHUNK):
        xs_ref[k] = xv[:, k * CHUNK:(k + 1) * CHUNK] * dv


def _conv1_body(a0_ref, a1_ref, x_ref, dinv_ref, w1_ref, b1_ref,
                h1_ref, hs_ref):
    a = jnp.concatenate([a0_ref[0] + a0_ref[1], a1_ref[0] + a1_ref[1]], axis=1)
    dv = dinv_ref[...]
    t = a * dv + x_ref[...] * (dv * dv)
    h1 = jnp.maximum(
        jnp.dot(t, w1_ref[...], preferred_element_type=jnp.float32)
        + b1_ref[...], 0.0)
    h1_ref[...] = h1
    hs = h1 * dv
    for k in range(HID // CHUNK):
        hs_ref[k] = hs[:, k * CHUNK:(k + 1) * CHUNK]


def _conv2_pool_body(a0_ref, a1_ref, a2_ref, a3_ref, h1_ref, dinv_ref,
                     batch_ref, w2_ref, b2_ref, gsum_ref, cnt_ref):
    i = pl.program_id(0)
    a = jnp.concatenate(
        [a0_ref[0] + a0_ref[1], a1_ref[0] + a1_ref[1],
         a2_ref[0] + a2_ref[1], a3_ref[0] + a3_ref[1]], axis=1)
    dv = dinv_ref[...]
    t = a * dv + h1_ref[...] * (dv * dv)
    h2 = jnp.maximum(
        jnp.dot(t, w2_ref[...], preferred_element_type=jnp.float32)
        + b2_ref[...], 0.0)
    gids = lax.broadcasted_iota(jnp.int32, (ROWBLK, NUM_GRAPHS), 1)
    oh = (batch_ref[...] == gids).astype(jnp.float32)
    gp = lax.dot_general(oh, h2, (((0,), (0,)), ((), ())),
                         preferred_element_type=jnp.float32)
    cp = jnp.sum(oh, axis=0, keepdims=True)

    @pl.when(i == 0)
    def _():
        gsum_ref[...] = jnp.zeros_like(gsum_ref)
        cnt_ref[...] = jnp.zeros_like(cnt_ref)

    gsum_ref[...] += gp
    cnt_ref[...] += cp


def _head_body(gsum_ref, cnt_ref, wmu_ref, bmu_ref, wlv_ref, blv_ref, eps_ref,
               wd1_ref, bd1_ref, wd2_ref, bd2_ref,
               mu_ref, lv_ref, h2_ref):
    cnt = jnp.maximum(cnt_ref[...], 1.0)
    g = gsum_ref[...] / jnp.transpose(cnt)
    mu = jnp.dot(g, wmu_ref[...], preferred_element_type=jnp.float32) + bmu_ref[...]
    lv = jnp.dot(g, wlv_ref[...], preferred_element_type=jnp.float32) + blv_ref[...]
    mu_ref[...] = mu
    lv_ref[...] = lv
    z = mu + jnp.exp(0.5 * lv) * eps_ref[...]
    h = jnp.maximum(jnp.dot(z, wd1_ref[...], preferred_element_type=jnp.float32)
                    + bd1_ref[...], 0.0)
    h = jnp.maximum(jnp.dot(h, wd2_ref[...], preferred_element_type=jnp.float32)
                    + bd2_ref[...], 0.0)
    h2_ref[...] = h


_LCOLS = 4096
_LROWS = _LCOLS // N_MAX


def _logits_body(h2_ref, wd3_ref, bd3_ref, out_ref):
    val = (jnp.dot(h2_ref[...], wd3_ref[...],
                   preferred_element_type=jnp.float32) + bd3_ref[...])
    out_ref[...] = val.reshape(NUM_GRAPHS, _LROWS, N_MAX)


def _sym_body(l_ref, eye_ref, out_ref):
    l = l_ref[...]
    lt = lax.dot_general(l, eye_ref[...], (((1,), (0,)), ((), ())),
                         preferred_element_type=jnp.float32)
    ls = (l + lt) * 0.5
    r = lax.broadcasted_iota(jnp.int32, (NUM_GRAPHS, N_MAX, N_MAX), 1)
    c = lax.broadcasted_iota(jnp.int32, (NUM_GRAPHS, N_MAX, N_MAX), 2)
    out_ref[...] = jnp.where(r == c, 0.0, 1.0 / (1.0 + jnp.exp(-ls)))


def kernel(x, edge_index, batch, W1, b1, W2, b2, Wmu, bmu, Wlv, blv,
           Wd1, bd1, Wd2, bd2, Wd3, bd3):
    f32 = jnp.float32
    src2 = edge_index[0].reshape(NW, STEPS, K)
    dst2 = edge_index[1].reshape(NW, STEPS, K)
    ones_k = jnp.ones((K, DEGW), f32)
    zeros_small = jnp.zeros((RPT, CHUNK), f32)

    degp = _deg_call(dst2, ones_k, zeros_small)

    xs_chunks, dinv = pl.pallas_call(
        _scale_body,
        grid=(NRB,),
        in_specs=[
            pl.BlockSpec((NC, ROWBLK, DEGW), lambda i: (0, i, 0)),
            pl.BlockSpec((ROWBLK, IN_DIM), lambda i: (i, 0)),
        ],
        out_specs=(
            pl.BlockSpec((IN_DIM // CHUNK, ROWBLK, CHUNK), lambda i: (0, i, 0)),
            pl.BlockSpec((ROWBLK, 1), lambda i: (i, 0)),
        ),
        out_shape=(
            jax.ShapeDtypeStruct((IN_DIM // CHUNK, N_NODES, CHUNK), f32),
            jax.ShapeDtypeStruct((N_NODES, 1), f32),
        ),
    )(degp, x)

    srcs = [src2 + jnp.int32(k * N_NODES) for k in range(HID // CHUNK)]
    xs_flat = xs_chunks.reshape((IN_DIM // CHUNK) * N_NODES, CHUNK)

    a1 = [_agg_call(xs_flat, srcs[k], dst2, zeros_small)
          for k in range(IN_DIM // CHUNK)]

    h1, hs_chunks = pl.pallas_call(
        _conv1_body,
        grid=(NRB,),
        in_specs=[
            pl.BlockSpec((NC, ROWBLK, CHUNK), lambda i: (0, i, 0)),
            pl.BlockSpec((NC, ROWBLK, CHUNK), lambda i: (0, i, 0)),
            pl.BlockSpec((ROWBLK, IN_DIM), lambda i: (i, 0)),
            pl.BlockSpec((ROWBLK, 1), lambda i: (i, 0)),
            pl.BlockSpec((IN_DIM, HID), lambda i: (0, 0)),
            pl.BlockSpec((HID,), lambda i: (0,)),
        ],
        out_specs=(
            pl.BlockSpec((ROWBLK, HID), lambda i: (i, 0)),
            pl.BlockSpec((HID // CHUNK, ROWBLK, CHUNK), lambda i: (0, i, 0)),
        ),
        out_shape=(
            jax.ShapeDtypeStruct((N_NODES, HID), f32),
            jax.ShapeDtypeStruct((HID // CHUNK, N_NODES, CHUNK), f32),
        ),
    )(a1[0], a1[1], x, dinv, W1, b1)

    hs_flat = hs_chunks.reshape((HID // CHUNK) * N_NODES, CHUNK)
    a2 = [_agg_call(hs_flat, srcs[k], dst2, zeros_small)
          for k in range(HID // CHUNK)]

    gsum, cnt = pl.pallas_call(
        _conv2_pool_body,
        grid=(NRB,),
        in_specs=[
            pl.BlockSpec((NC, ROWBLK, CHUNK), lambda i: (0, i, 0)),
            pl.BlockSpec((NC, ROWBLK, CHUNK), lambda i: (0, i, 0)),
            pl.BlockSpec((NC, ROWBLK, CHUNK), lambda i: (0, i, 0)),
            pl.BlockSpec((NC, ROWBLK, CHUNK), lambda i: (0, i, 0)),
            pl.BlockSpec((ROWBLK, HID), lambda i: (i, 0)),
            pl.BlockSpec((ROWBLK, 1), lambda i: (i, 0)),
            pl.BlockSpec((ROWBLK, 1), lambda i: (i, 0)),
            pl.BlockSpec((HID, HID), lambda i: (0, 0)),
            pl.BlockSpec((HID,), lambda i: (0,)),
        ],
        out_specs=(
            pl.BlockSpec((NUM_GRAPHS, HID), lambda i: (0, 0)),
            pl.BlockSpec((1, NUM_GRAPHS), lambda i: (0, 0)),
        ),
        out_shape=(
            jax.ShapeDtypeStruct((NUM_GRAPHS, HID), f32),
            jax.ShapeDtypeStruct((1, NUM_GRAPHS), f32),
        ),
    )(a2[0], a2[1], a2[2], a2[3], h1, dinv, batch[:, None], W2, b2)

    eps = jax.random.normal(jax.random.key(42), (NUM_GRAPHS, LAT), dtype=f32)

    mu, logv, hdec = pl.pallas_call(
        _head_body,
        out_shape=(
            jax.ShapeDtypeStruct((NUM_GRAPHS, LAT), f32),
            jax.ShapeDtypeStruct((NUM_GRAPHS, LAT), f32),
            jax.ShapeDtypeStruct((NUM_GRAPHS, HID), f32),
        ),
    )(gsum, cnt, Wmu, bmu, Wlv, blv, eps, Wd1, bd1, Wd2, bd2)

    logits = pl.pallas_call(
        _logits_body,
        grid=((N_MAX * N_MAX) // _LCOLS,),
        in_specs=[
            pl.BlockSpec((NUM_GRAPHS, HID), lambda i: (0, 0)),
            pl.BlockSpec((HID, _LCOLS), lambda i: (0, i)),
            pl.BlockSpec((_LCOLS,), lambda i: (i,)),
        ],
        out_specs=pl.BlockSpec((NUM_GRAPHS, _LROWS, N_MAX), lambda i: (0, i, 0)),
        out_shape=jax.ShapeDtypeStruct((NUM_GRAPHS, N_MAX, N_MAX), f32),
    )(hdec, Wd3, bd3)

    probs = pl.pallas_call(
        _sym_body,
        out_shape=jax.ShapeDtypeStruct((NUM_GRAPHS, N_MAX, N_MAX), f32),
    )(logits, jnp.eye(N_MAX, dtype=f32))

    return (probs, mu, logv)


# speedup vs baseline: 1.1487x; 1.0069x over previous
import functools

import jax
import jax.numpy as jnp
from jax import lax
from jax.experimental import pallas as pl
from jax.experimental.pallas import tpu as pltpu
from jax.experimental.pallas import tpu_sc as plsc

N_NODES = 10000
N_EDGES = 160000
IN_DIM = 256
HID = 512
LAT = 128
N_MAX = 128
NUM_GRAPHS = 64

NC = 2
NS = 16
NW = NC * NS
EPT = N_EDGES // NW
K = 125
STEPS = EPT // K
CHUNK = 128
RPT = 624
RTAIL = N_NODES - NS * RPT
RTOFF = NS * RPT

ROWBLK = 2000
NRB = N_NODES // ROWBLK


DEGW = 128


def _deg_sc_body(dstw, ones_h, zeros_h, out_h, idxd_v, ones_v, acc_sh):
    c = lax.axis_index("c")
    s = lax.axis_index("s")
    w = c * NS + s
    pltpu.sync_copy(dstw.at[w], idxd_v)
    pltpu.sync_copy(ones_h, ones_v)
    r0 = s * RPT
    pltpu.sync_copy(zeros_h.at[pl.ds(r0, RPT)], acc_sh.at[pl.ds(r0, RPT)])

    @pl.when(s == 0)
    def _():
        pltpu.sync_copy(zeros_h.at[pl.ds(RTOFF, RTAIL)],
                        acc_sh.at[pl.ds(RTOFF, RTAIL)])

    plsc.subcore_barrier()

    def step(j, carry):
        pltpu.sync_copy(ones_v, acc_sh.at[idxd_v.at[j]], add=True)
        return carry

    lax.fori_loop(0, STEPS, step, 0)
    plsc.subcore_barrier()
    pltpu.sync_copy(acc_sh.at[pl.ds(r0, RPT)], out_h.at[c].at[pl.ds(r0, RPT)])

    @pl.when(s == 0)
    def _():
        pltpu.sync_copy(acc_sh.at[pl.ds(RTOFF, RTAIL)],
                        out_h.at[c].at[pl.ds(RTOFF, RTAIL)])


_deg_call = pl.kernel(
    _deg_sc_body,
    out_type=jax.ShapeDtypeStruct((NC, N_NODES, DEGW), jnp.float32),
    mesh=plsc.VectorSubcoreMesh(core_axis_name="c", subcore_axis_name="s"),
    scratch_types=[
        pltpu.VMEM((STEPS, K), jnp.int32),
        pltpu.VMEM((K, DEGW), jnp.float32),
        pltpu.VMEM_SHARED((N_NODES, DEGW), jnp.float32),
    ],
)


NPAIR = STEPS // 2


def _agg_sc_body(xk_h, srcw, dstw, zeros_h, out_h, idxs_v, idxd_v,
                 buf0, buf1, sem0, sem1, acc_sh):
    c = lax.axis_index("c")
    s = lax.axis_index("s")
    w = c * NS + s
    pltpu.sync_copy(srcw.at[w], idxs_v)
    pltpu.sync_copy(dstw.at[w], idxd_v)
    r0 = s * RPT
    pltpu.sync_copy(zeros_h.at[pl.ds(r0, RPT)], acc_sh.at[pl.ds(r0, RPT)])

    @pl.when(s == 0)
    def _():
        pltpu.sync_copy(zeros_h.at[pl.ds(RTOFF, RTAIL)],
                        acc_sh.at[pl.ds(RTOFF, RTAIL)])

    plsc.subcore_barrier()
    pltpu.async_copy(xk_h.at[idxs_v.at[0]], buf0, sem0)

    def pair(j2, carry):
        j = j2 * 2
        pltpu.make_async_copy(xk_h.at[idxs_v.at[j]], buf0, sem0).wait()
        pltpu.async_copy(xk_h.at[idxs_v.at[j + 1]], buf1, sem1)
        pltpu.sync_copy(buf0, acc_sh.at[idxd_v.at[j]], add=True)
        pltpu.make_async_copy(xk_h.at[idxs_v.at[j + 1]], buf1, sem1).wait()

        @pl.when(j2 + 1 < NPAIR)
        def _():
            pltpu.async_copy(xk_h.at[idxs_v.at[j + 2]], buf0, sem0)

        pltpu.sync_copy(buf1, acc_sh.at[idxd_v.at[j + 1]], add=True)
        return carry

    lax.fori_loop(0, NPAIR, pair, 0)
    plsc.subcore_barrier()
    pltpu.sync_copy(acc_sh.at[pl.ds(r0, RPT)], out_h.at[c].at[pl.ds(r0, RPT)])

    @pl.when(s == 0)
    def _():
        pltpu.sync_copy(acc_sh.at[pl.ds(RTOFF, RTAIL)],
                        out_h.at[c].at[pl.ds(RTOFF, RTAIL)])


_agg_call = pl.kernel(
    _agg_sc_body,
    out_type=jax.ShapeDtypeStruct((NC, N_NODES, CHUNK), jnp.float32),
    mesh=plsc.VectorSubcoreMesh(core_axis_name="c", subcore_axis_name="s"),
    scratch_types=[
        pltpu.VMEM((STEPS, K), jnp.int32),
        pltpu.VMEM((STEPS, K), jnp.int32),
        pltpu.VMEM((K, CHUNK), jnp.float32),
        pltpu.VMEM((K, CHUNK), jnp.float32),
        pltpu.SemaphoreType.DMA,
        pltpu.SemaphoreType.DMA,
        pltpu.VMEM_SHARED((N_NODES, CHUNK), jnp.float32),
    ],
)


def _scale_body(degp_ref, x_ref, xs_ref, dinv_ref):
    p = degp_ref[...]
    dv = lax.rsqrt(1.0 + p[0, :, 0:1] + p[1, :, 0:1])
    dinv_ref[...] = dv
    xv = x_ref[...]
    for k in range(IN_DIM // CHUNK):
        xs_ref[k] = xv[:, k * CHUNK:(k + 1) * CHUNK] * dv


def _conv1_body(a0_ref, a1_ref, x_ref, dinv_ref, w1_ref, b1_ref,
                h1_ref, hs_ref):
    a = jnp.concatenate([a0_ref[0] + a0_ref[1], a1_ref[0] + a1_ref[1]], axis=1)
    dv = dinv_ref[...]
    t = a * dv + x_ref[...] * (dv * dv)
    h1 = jnp.maximum(
        jnp.dot(t, w1_ref[...], preferred_element_type=jnp.float32)
        + b1_ref[...], 0.0)
    h1_ref[...] = h1
    hs = h1 * dv
    for k in range(HID // CHUNK):
        hs_ref[k] = hs[:, k * CHUNK:(k + 1) * CHUNK]


def _conv2_pool_body(a0_ref, a1_ref, a2_ref, a3_ref, h1_ref, dinv_ref,
                     batch_ref, w2_ref, b2_ref, gsum_ref, cnt_ref):
    i = pl.program_id(0)
    a = jnp.concatenate(
        [a0_ref[0] + a0_ref[1], a1_ref[0] + a1_ref[1],
         a2_ref[0] + a2_ref[1], a3_ref[0] + a3_ref[1]], axis=1)
    dv = dinv_ref[...]
    t = a * dv + h1_ref[...] * (dv * dv)
    h2 = jnp.maximum(
        jnp.dot(t, w2_ref[...], preferred_element_type=jnp.float32)
        + b2_ref[...], 0.0)
    gids = lax.broadcasted_iota(jnp.int32, (ROWBLK, NUM_GRAPHS), 1)
    oh = (batch_ref[...] == gids).astype(jnp.float32)
    gp = lax.dot_general(oh, h2, (((0,), (0,)), ((), ())),
                         preferred_element_type=jnp.float32)
    cp = jnp.sum(oh, axis=0, keepdims=True)

    @pl.when(i == 0)
    def _():
        gsum_ref[...] = jnp.zeros_like(gsum_ref)
        cnt_ref[...] = jnp.zeros_like(cnt_ref)

    gsum_ref[...] += gp
    cnt_ref[...] += cp


def _head_body(gsum_ref, cnt_ref, wmu_ref, bmu_ref, wlv_ref, blv_ref, eps_ref,
               wd1_ref, bd1_ref, wd2_ref, bd2_ref,
               mu_ref, lv_ref, h2_ref):
    cnt = jnp.maximum(cnt_ref[...], 1.0)
    g = gsum_ref[...] / jnp.transpose(cnt)
    mu = jnp.dot(g, wmu_ref[...], preferred_element_type=jnp.float32) + bmu_ref[...]
    lv = jnp.dot(g, wlv_ref[...], preferred_element_type=jnp.float32) + blv_ref[...]
    mu_ref[...] = mu
    lv_ref[...] = lv
    z = mu + jnp.exp(0.5 * lv) * eps_ref[...]
    h = jnp.maximum(jnp.dot(z, wd1_ref[...], preferred_element_type=jnp.float32)
                    + bd1_ref[...], 0.0)
    h = jnp.maximum(jnp.dot(h, wd2_ref[...], preferred_element_type=jnp.float32)
                    + bd2_ref[...], 0.0)
    h2_ref[...] = h


_LCOLS = 4096
_LROWS = _LCOLS // N_MAX


def _logits_body(h2_ref, wd3_ref, bd3_ref, out_ref):
    val = (jnp.dot(h2_ref[...], wd3_ref[...],
                   preferred_element_type=jnp.float32) + bd3_ref[...])
    out_ref[...] = val.reshape(NUM_GRAPHS, _LROWS, N_MAX)


def _sym_body(l_ref, eye_ref, out_ref):
    l = l_ref[...]
    lt = lax.dot_general(l, eye_ref[...], (((1,), (0,)), ((), ())),
                         preferred_element_type=jnp.float32)
    ls = (l + lt) * 0.5
    r = lax.broadcasted_iota(jnp.int32, (NUM_GRAPHS, N_MAX, N_MAX), 1)
    c = lax.broadcasted_iota(jnp.int32, (NUM_GRAPHS, N_MAX, N_MAX), 2)
    out_ref[...] = jnp.where(r == c, 0.0, 1.0 / (1.0 + jnp.exp(-ls)))


def kernel(x, edge_index, batch, W1, b1, W2, b2, Wmu, bmu, Wlv, blv,
           Wd1, bd1, Wd2, bd2, Wd3, bd3):
    f32 = jnp.float32
    src2 = edge_index[0].reshape(NW, STEPS, K)
    dst2 = edge_index[1].reshape(NW, STEPS, K)
    ones_k = jnp.ones((K, DEGW), f32)
    zeros_small = jnp.zeros((N_NODES, CHUNK), f32)

    degp = _deg_call(dst2, ones_k, zeros_small)

    xs_chunks, dinv = pl.pallas_call(
        _scale_body,
        grid=(NRB,),
        in_specs=[
            pl.BlockSpec((NC, ROWBLK, DEGW), lambda i: (0, i, 0)),
            pl.BlockSpec((ROWBLK, IN_DIM), lambda i: (i, 0)),
        ],
        out_specs=(
            pl.BlockSpec((IN_DIM // CHUNK, ROWBLK, CHUNK), lambda i: (0, i, 0)),
            pl.BlockSpec((ROWBLK, 1), lambda i: (i, 0)),
        ),
        out_shape=(
            jax.ShapeDtypeStruct((IN_DIM // CHUNK, N_NODES, CHUNK), f32),
            jax.ShapeDtypeStruct((N_NODES, 1), f32),
        ),
    )(degp, x)

    srcs = [src2 + jnp.int32(k * N_NODES) for k in range(HID // CHUNK)]
    xs_flat = xs_chunks.reshape((IN_DIM // CHUNK) * N_NODES, CHUNK)

    a1 = [_agg_call(xs_flat, srcs[k], dst2, zeros_small)
          for k in range(IN_DIM // CHUNK)]

    h1, hs_chunks = pl.pallas_call(
        _conv1_body,
        grid=(NRB,),
        in_specs=[
            pl.BlockSpec((NC, ROWBLK, CHUNK), lambda i: (0, i, 0)),
            pl.BlockSpec((NC, ROWBLK, CHUNK), lambda i: (0, i, 0)),
            pl.BlockSpec((ROWBLK, IN_DIM), lambda i: (i, 0)),
            pl.BlockSpec((ROWBLK, 1), lambda i: (i, 0)),
            pl.BlockSpec((IN_DIM, HID), lambda i: (0, 0)),
            pl.BlockSpec((HID,), lambda i: (0,)),
        ],
        out_specs=(
            pl.BlockSpec((ROWBLK, HID), lambda i: (i, 0)),
            pl.BlockSpec((HID // CHUNK, ROWBLK, CHUNK), lambda i: (0, i, 0)),
        ),
        out_shape=(
            jax.ShapeDtypeStruct((N_NODES, HID), f32),
            jax.ShapeDtypeStruct((HID // CHUNK, N_NODES, CHUNK), f32),
        ),
    )(a1[0], a1[1], x, dinv, W1, b1)

    hs_flat = hs_chunks.reshape((HID // CHUNK) * N_NODES, CHUNK)
    a2 = [_agg_call(hs_flat, srcs[k], dst2, zeros_small)
          for k in range(HID // CHUNK)]

    gsum, cnt = pl.pallas_call(
        _conv2_pool_body,
        grid=(NRB,),
        in_specs=[
            pl.BlockSpec((NC, ROWBLK, CHUNK), lambda i: (0, i, 0)),
            pl.BlockSpec((NC, ROWBLK, CHUNK), lambda i: (0, i, 0)),
            pl.BlockSpec((NC, ROWBLK, CHUNK), lambda i: (0, i, 0)),
            pl.BlockSpec((NC, ROWBLK, CHUNK), lambda i: (0, i, 0)),
            pl.BlockSpec((ROWBLK, HID), lambda i: (i, 0)),
            pl.BlockSpec((ROWBLK, 1), lambda i: (i, 0)),
            pl.BlockSpec((ROWBLK, 1), lambda i: (i, 0)),
            pl.BlockSpec((HID, HID), lambda i: (0, 0)),
            pl.BlockSpec((HID,), lambda i: (0,)),
        ],
        out_specs=(
            pl.BlockSpec((NUM_GRAPHS, HID), lambda i: (0, 0)),
            pl.BlockSpec((1, NUM_GRAPHS), lambda i: (0, 0)),
        ),
        out_shape=(
            jax.ShapeDtypeStruct((NUM_GRAPHS, HID), f32),
            jax.ShapeDtypeStruct((1, NUM_GRAPHS), f32),
        ),
    )(a2[0], a2[1], a2[2], a2[3], h1, dinv, batch[:, None], W2, b2)

    eps = jax.random.normal(jax.random.key(42), (NUM_GRAPHS, LAT), dtype=f32)

    mu, logv, hdec = pl.pallas_call(
        _head_body,
        out_shape=(
            jax.ShapeDtypeStruct((NUM_GRAPHS, LAT), f32),
            jax.ShapeDtypeStruct((NUM_GRAPHS, LAT), f32),
            jax.ShapeDtypeStruct((NUM_GRAPHS, HID), f32),
        ),
    )(gsum, cnt, Wmu, bmu, Wlv, blv, eps, Wd1, bd1, Wd2, bd2)

    logits = pl.pallas_call(
        _logits_body,
        grid=((N_MAX * N_MAX) // _LCOLS,),
        in_specs=[
            pl.BlockSpec((NUM_GRAPHS, HID), lambda i: (0, 0)),
            pl.BlockSpec((HID, _LCOLS), lambda i: (0, i)),
            pl.BlockSpec((_LCOLS,), lambda i: (i,)),
        ],
        out_specs=pl.BlockSpec((NUM_GRAPHS, _LROWS, N_MAX), lambda i: (0, i, 0)),
        out_shape=jax.ShapeDtypeStruct((NUM_GRAPHS, N_MAX, N_MAX), f32),
    )(hdec, Wd3, bd3)

    probs = pl.pallas_call(
        _sym_body,
        out_shape=jax.ShapeDtypeStruct((NUM_GRAPHS, N_MAX, N_MAX), f32),
    )(logits, jnp.eye(N_MAX, dtype=f32))

    return (probs, mu, logv)


# speedup vs baseline: 1.1523x; 1.0031x over previous
import jax
import jax.numpy as jnp
from jax import lax
from jax.experimental import pallas as pl
from jax.experimental.pallas import tpu as pltpu
from jax.experimental.pallas import tpu_sc as plsc

N_NODES = 10000
N_EDGES = 160000
IN_DIM = 256
HID = 512
LAT = 128
N_MAX = 128
NUM_GRAPHS = 64

NC = 2
NS = 16
NW = NC * NS
EPT = N_EDGES // NW
K = 125
STEPS = EPT // K
CHUNK = 128
RPT = 624
RTAIL = N_NODES - NS * RPT
RTOFF = NS * RPT

ROWBLK = 2000
NRB = N_NODES // ROWBLK


DEGW = 128


def _deg_sc_body(dstw, ones_h, zeros_h, out_h, idxd_v, ones_v, acc_sh):
    c = lax.axis_index("c")
    s = lax.axis_index("s")
    w = c * NS + s
    pltpu.sync_copy(dstw.at[w], idxd_v)
    pltpu.sync_copy(ones_h, ones_v)
    r0 = s * RPT
    pltpu.sync_copy(zeros_h.at[pl.ds(r0, RPT)], acc_sh.at[pl.ds(r0, RPT)])

    @pl.when(s == 0)
    def _():
        pltpu.sync_copy(zeros_h.at[pl.ds(RTOFF, RTAIL)],
                        acc_sh.at[pl.ds(RTOFF, RTAIL)])

    plsc.subcore_barrier()

    def step(j, carry):
        pltpu.sync_copy(ones_v, acc_sh.at[idxd_v.at[j]], add=True)
        return carry

    lax.fori_loop(0, STEPS, step, 0)
    plsc.subcore_barrier()
    pltpu.sync_copy(acc_sh.at[pl.ds(r0, RPT)], out_h.at[c].at[pl.ds(r0, RPT)])

    @pl.when(s == 0)
    def _():
        pltpu.sync_copy(acc_sh.at[pl.ds(RTOFF, RTAIL)],
                        out_h.at[c].at[pl.ds(RTOFF, RTAIL)])


_deg_call = pl.kernel(
    _deg_sc_body,
    out_type=jax.ShapeDtypeStruct((NC, N_NODES, DEGW), jnp.float32),
    mesh=plsc.VectorSubcoreMesh(core_axis_name="c", subcore_axis_name="s"),
    scratch_types=[
        pltpu.VMEM((STEPS, K), jnp.int32),
        pltpu.VMEM((K, DEGW), jnp.float32),
        pltpu.VMEM_SHARED((N_NODES, DEGW), jnp.float32),
    ],
)


NPAIR = STEPS // 2


def _agg_sc_body(xk_h, srcw, dstw, zeros_h, out_h, idxs_v, idxd_v,
                 buf0, buf1, sem0, sem1, acc_sh):
    c = lax.axis_index("c")
    s = lax.axis_index("s")
    w = c * NS + s
    pltpu.sync_copy(srcw.at[w], idxs_v)
    pltpu.sync_copy(dstw.at[w], idxd_v)
    r0 = s * RPT
    pltpu.sync_copy(zeros_h.at[pl.ds(r0, RPT)], acc_sh.at[pl.ds(r0, RPT)])

    @pl.when(s == 0)
    def _():
        pltpu.sync_copy(zeros_h.at[pl.ds(RTOFF, RTAIL)],
                        acc_sh.at[pl.ds(RTOFF, RTAIL)])

    plsc.subcore_barrier()
    pltpu.async_copy(xk_h.at[idxs_v.at[0]], buf0, sem0)

    def pair(j2, carry):
        j = j2 * 2
        pltpu.make_async_copy(xk_h.at[idxs_v.at[j]], buf0, sem0).wait()
        pltpu.async_copy(xk_h.at[idxs_v.at[j + 1]], buf1, sem1)
        pltpu.sync_copy(buf0, acc_sh.at[idxd_v.at[j]], add=True)
        pltpu.make_async_copy(xk_h.at[idxs_v.at[j + 1]], buf1, sem1).wait()

        @pl.when(j2 + 1 < NPAIR)
        def _():
            pltpu.async_copy(xk_h.at[idxs_v.at[j + 2]], buf0, sem0)

        pltpu.sync_copy(buf1, acc_sh.at[idxd_v.at[j + 1]], add=True)
        return carry

    lax.fori_loop(0, NPAIR, pair, 0)
    plsc.subcore_barrier()
    pltpu.sync_copy(acc_sh.at[pl.ds(r0, RPT)], out_h.at[c].at[pl.ds(r0, RPT)])

    @pl.when(s == 0)
    def _():
        pltpu.sync_copy(acc_sh.at[pl.ds(RTOFF, RTAIL)],
                        out_h.at[c].at[pl.ds(RTOFF, RTAIL)])


_agg_call = pl.kernel(
    _agg_sc_body,
    out_type=jax.ShapeDtypeStruct((NC, N_NODES, CHUNK), jnp.float32),
    mesh=plsc.VectorSubcoreMesh(core_axis_name="c", subcore_axis_name="s"),
    scratch_types=[
        pltpu.VMEM((STEPS, K), jnp.int32),
        pltpu.VMEM((STEPS, K), jnp.int32),
        pltpu.VMEM((K, CHUNK), jnp.float32),
        pltpu.VMEM((K, CHUNK), jnp.float32),
        pltpu.SemaphoreType.DMA,
        pltpu.SemaphoreType.DMA,
        pltpu.VMEM_SHARED((N_NODES, CHUNK), jnp.float32),
    ],
)


def _scale_body(degp_ref, x_ref, xs_ref, dinv_ref):
    p = degp_ref[...]
    dv = lax.rsqrt(1.0 + p[0, :, 0:1] + p[1, :, 0:1])
    dinv_ref[...] = dv
    xv = x_ref[...]
    for k in range(IN_DIM // CHUNK):
        xs_ref[k] = xv[:, k * CHUNK:(k + 1) * CHUNK] * dv


def _conv1_body(a0_ref, a1_ref, x_ref, dinv_ref, w1_ref, b1_ref,
                h1_ref, hs_ref):
    a = jnp.concatenate([a0_ref[0] + a0_ref[1], a1_ref[0] + a1_ref[1]], axis=1)
    dv = dinv_ref[...]
    t = a * dv + x_ref[...] * (dv * dv)
    h1 = jnp.maximum(
        jnp.dot(t, w1_ref[...], preferred_element_type=jnp.float32)
        + b1_ref[...], 0.0)
    h1_ref[...] = h1
    hs = h1 * dv
    for k in range(HID // CHUNK):
        hs_ref[k] = hs[:, k * CHUNK:(k + 1) * CHUNK]


def _conv2_pool_body(a0_ref, a1_ref, a2_ref, a3_ref, h1_ref, dinv_ref,
                     batch_ref, w2_ref, b2_ref, gsum_ref, cnt_ref):
    i = pl.program_id(0)
    a = jnp.concatenate(
        [a0_ref[0] + a0_ref[1], a1_ref[0] + a1_ref[1],
         a2_ref[0] + a2_ref[1], a3_ref[0] + a3_ref[1]], axis=1)
    dv = dinv_ref[...]
    t = a * dv + h1_ref[...] * (dv * dv)
    h2 = jnp.maximum(
        jnp.dot(t, w2_ref[...], preferred_element_type=jnp.float32)
        + b2_ref[...], 0.0)
    gids = lax.broadcasted_iota(jnp.int32, (ROWBLK, NUM_GRAPHS), 1)
    oh = (batch_ref[...] == gids).astype(jnp.float32)
    gp = lax.dot_general(oh, h2, (((0,), (0,)), ((), ())),
                         preferred_element_type=jnp.float32)
    cp = jnp.sum(oh, axis=0, keepdims=True)

    @pl.when(i == 0)
    def _():
        gsum_ref[...] = jnp.zeros_like(gsum_ref)
        cnt_ref[...] = jnp.zeros_like(cnt_ref)

    gsum_ref[...] += gp
    cnt_ref[...] += cp


def _head_body(gsum_ref, cnt_ref, wmu_ref, bmu_ref, wlv_ref, blv_ref, eps_ref,
               wd1_ref, bd1_ref, wd2_ref, bd2_ref,
               mu_ref, lv_ref, h2_ref):
    cnt = jnp.maximum(cnt_ref[...], 1.0)
    g = gsum_ref[...] / jnp.transpose(cnt)
    mu = jnp.dot(g, wmu_ref[...], preferred_element_type=jnp.float32) + bmu_ref[...]
    lv = jnp.dot(g, wlv_ref[...], preferred_element_type=jnp.float32) + blv_ref[...]
    mu_ref[...] = mu
    lv_ref[...] = lv
    z = mu + jnp.exp(0.5 * lv) * eps_ref[...]
    h = jnp.maximum(jnp.dot(z, wd1_ref[...], preferred_element_type=jnp.float32)
                    + bd1_ref[...], 0.0)
    h = jnp.maximum(jnp.dot(h, wd2_ref[...], preferred_element_type=jnp.float32)
                    + bd2_ref[...], 0.0)
    h2_ref[...] = h


_LCOLS = 4096
_LROWS = _LCOLS // N_MAX


def _logits_body(h2_ref, wd3_ref, bd3_ref, out_ref):
    val = (jnp.dot(h2_ref[...], wd3_ref[...],
                   preferred_element_type=jnp.float32) + bd3_ref[...])
    out_ref[...] = val.reshape(NUM_GRAPHS, _LROWS, N_MAX)


def _sym_body(l_ref, eye_ref, out_ref):
    l = l_ref[...]
    lt = lax.dot_general(l, eye_ref[...], (((1,), (0,)), ((), ())),
                         preferred_element_type=jnp.float32)
    ls = (l + lt) * 0.5
    r = lax.broadcasted_iota(jnp.int32, (NUM_GRAPHS, N_MAX, N_MAX), 1)
    c = lax.broadcasted_iota(jnp.int32, (NUM_GRAPHS, N_MAX, N_MAX), 2)
    out_ref[...] = jnp.where(r == c, 0.0, 1.0 / (1.0 + jnp.exp(-ls)))


def kernel(x, edge_index, batch, W1, b1, W2, b2, Wmu, bmu, Wlv, blv,
           Wd1, bd1, Wd2, bd2, Wd3, bd3):
    f32 = jnp.float32
    src2 = edge_index[0].reshape(NW, STEPS, K)
    dst2 = edge_index[1].reshape(NW, STEPS, K)
    ones_k = jnp.ones((K, DEGW), f32)
    zeros_small = jnp.zeros((N_NODES, CHUNK), f32)

    degp = _deg_call(dst2, ones_k, zeros_small)

    xs_chunks, dinv = pl.pallas_call(
        _scale_body,
        grid=(NRB,),
        in_specs=[
            pl.BlockSpec((NC, ROWBLK, DEGW), lambda i: (0, i, 0)),
            pl.BlockSpec((ROWBLK, IN_DIM), lambda i: (i, 0)),
        ],
        out_specs=(
            pl.BlockSpec((IN_DIM // CHUNK, ROWBLK, CHUNK), lambda i: (0, i, 0)),
            pl.BlockSpec((ROWBLK, 1), lambda i: (i, 0)),
        ),
        out_shape=(
            jax.ShapeDtypeStruct((IN_DIM // CHUNK, N_NODES, CHUNK), f32),
            jax.ShapeDtypeStruct((N_NODES, 1), f32),
        ),
    )(degp, x)

    srcs = [src2 + jnp.int32(k * N_NODES) for k in range(HID // CHUNK)]
    xs_flat = xs_chunks.reshape((IN_DIM // CHUNK) * N_NODES, CHUNK)

    a1 = [_agg_call(xs_flat, srcs[k], dst2, zeros_small)
          for k in range(IN_DIM // CHUNK)]

    h1, hs_chunks = pl.pallas_call(
        _conv1_body,
        grid=(NRB,),
        in_specs=[
            pl.BlockSpec((NC, ROWBLK, CHUNK), lambda i: (0, i, 0)),
            pl.BlockSpec((NC, ROWBLK, CHUNK), lambda i: (0, i, 0)),
            pl.BlockSpec((ROWBLK, IN_DIM), lambda i: (i, 0)),
            pl.BlockSpec((ROWBLK, 1), lambda i: (i, 0)),
            pl.BlockSpec((IN_DIM, HID), lambda i: (0, 0)),
            pl.BlockSpec((HID,), lambda i: (0,)),
        ],
        out_specs=(
            pl.BlockSpec((ROWBLK, HID), lambda i: (i, 0)),
            pl.BlockSpec((HID // CHUNK, ROWBLK, CHUNK), lambda i: (0, i, 0)),
        ),
        out_shape=(
            jax.ShapeDtypeStruct((N_NODES, HID), f32),
            jax.ShapeDtypeStruct((HID // CHUNK, N_NODES, CHUNK), f32),
        ),
    )(a1[0], a1[1], x, dinv, W1, b1)

    hs_flat = hs_chunks.reshape((HID // CHUNK) * N_NODES, CHUNK)
    a2 = [_agg_call(hs_flat, srcs[k], dst2, zeros_small)
          for k in range(HID // CHUNK)]

    gsum, cnt = pl.pallas_call(
        _conv2_pool_body,
        grid=(NRB,),
        in_specs=[
            pl.BlockSpec((NC, ROWBLK, CHUNK), lambda i: (0, i, 0)),
            pl.BlockSpec((NC, ROWBLK, CHUNK), lambda i: (0, i, 0)),
            pl.BlockSpec((NC, ROWBLK, CHUNK), lambda i: (0, i, 0)),
            pl.BlockSpec((NC, ROWBLK, CHUNK), lambda i: (0, i, 0)),
            pl.BlockSpec((ROWBLK, HID), lambda i: (i, 0)),
            pl.BlockSpec((ROWBLK, 1), lambda i: (i, 0)),
            pl.BlockSpec((ROWBLK, 1), lambda i: (i, 0)),
            pl.BlockSpec((HID, HID), lambda i: (0, 0)),
            pl.BlockSpec((HID,), lambda i: (0,)),
        ],
        out_specs=(
            pl.BlockSpec((NUM_GRAPHS, HID), lambda i: (0, 0)),
            pl.BlockSpec((1, NUM_GRAPHS), lambda i: (0, 0)),
        ),
        out_shape=(
            jax.ShapeDtypeStruct((NUM_GRAPHS, HID), f32),
            jax.ShapeDtypeStruct((1, NUM_GRAPHS), f32),
        ),
    )(a2[0], a2[1], a2[2], a2[3], h1, dinv, batch[:, None], W2, b2)

    eps = jax.random.normal(jax.random.key(42), (NUM_GRAPHS, LAT), dtype=f32)

    mu, logv, hdec = pl.pallas_call(
        _head_body,
        out_shape=(
            jax.ShapeDtypeStruct((NUM_GRAPHS, LAT), f32),
            jax.ShapeDtypeStruct((NUM_GRAPHS, LAT), f32),
            jax.ShapeDtypeStruct((NUM_GRAPHS, HID), f32),
        ),
    )(gsum, cnt, Wmu, bmu, Wlv, blv, eps, Wd1, bd1, Wd2, bd2)

    logits = pl.pallas_call(
        _logits_body,
        grid=((N_MAX * N_MAX) // _LCOLS,),
        in_specs=[
            pl.BlockSpec((NUM_GRAPHS, HID), lambda i: (0, 0)),
            pl.BlockSpec((HID, _LCOLS), lambda i: (0, i)),
            pl.BlockSpec((_LCOLS,), lambda i: (i,)),
        ],
        out_specs=pl.BlockSpec((NUM_GRAPHS, _LROWS, N_MAX), lambda i: (0, i, 0)),
        out_shape=jax.ShapeDtypeStruct((NUM_GRAPHS, N_MAX, N_MAX), f32),
    )(hdec, Wd3, bd3)

    probs = pl.pallas_call(
        _sym_body,
        out_shape=jax.ShapeDtypeStruct((NUM_GRAPHS, N_MAX, N_MAX), f32),
    )(logits, jnp.eye(N_MAX, dtype=f32))

    return (probs, mu, logv)
